# baseline (speedup 1.0000x reference)
import jax
import jax.numpy as jnp
from jax import lax
from jax.experimental import pallas as pl
from jax.experimental.pallas import tpu as pltpu
from jax.experimental.pallas import tpu_sc as plsc
from jax._src.pallas import mpmd as _mpmd

DEPTH = 1000
B_CONST = 1024
N_WORDS = 50 * DEPTH * B_CONST
FILL_CHUNK = 640000
FILL_NBUF = 8
L = 16


def _fill_body(off_ref, out_ref, *scratch):
    bufs = scratch[:FILL_NBUF]
    sems = scratch[FILL_NBUF:]
    off = off_ref[0, 0]
    for k in range(FILL_NBUF):
        bufs[k][...] = jnp.full((FILL_CHUNK,), off, jnp.float32)

    n_outer = N_WORDS // FILL_CHUNK // FILL_NBUF

    def outer(i, carry):
        for k in range(FILL_NBUF):
            c = i * FILL_NBUF + k

            @pl.when(i > 0)
            def _wait():
                pltpu.make_async_copy(
                    bufs[k], out_ref.at[pl.ds(c * FILL_CHUNK, FILL_CHUNK)], sems[k]
                ).wait()

            pltpu.make_async_copy(
                bufs[k], out_ref.at[pl.ds(c * FILL_CHUNK, FILL_CHUNK)], sems[k]
            ).start()
        return carry

    lax.fori_loop(0, n_outer, outer, 0)
    for k in range(FILL_NBUF):
        pltpu.make_async_copy(
            bufs[k], out_ref.at[pl.ds(0, FILL_CHUNK)], sems[k]
        ).wait()


def _scatter_body(filled_hbm, xt_hbm, on_hbm, out_hbm, xblk_v, onbuf_v, idx_v):
    del filled_hbm
    info = plsc.get_sparse_core_info()
    nc = info.num_cores
    s, b = xt_hbm.shape
    wid = lax.axis_index("s") * nc + lax.axis_index("c")
    iblk = b // (nc * info.num_subcores)

    pltpu.sync_copy(xt_hbm.at[:, pl.ds(wid * iblk, iblk)], xblk_v)
    pltpu.sync_copy(on_hbm, onbuf_v)
    lanes = lax.iota(jnp.int32, L)

    def build(j, _):
        for k in range(2):
            v = xblk_v[j, pl.ds(k * L, L)]
            base = wid * iblk + k * L
            scal = j * (DEPTH * B_CONST) + (base >> 7) * 1024 + (base & 127)
            idx16 = ((v >> 3) << 13) + ((v & 7) << 7) + (scal + lanes)
            p = j * iblk + k * L
            idx_v[p >> 6, pl.ds(p & 63, L)] = idx16
        return 0

    lax.fori_loop(0, s, build, 0)

    def run_scoped_sem(sem):
        n_fires = (s * iblk) // 64
        for r in range(n_fires):
            pltpu.make_async_copy(onbuf_v, out_hbm.at[idx_v.at[r]], sem).start()
        for r in range(n_fires):
            pltpu.make_async_copy(onbuf_v, out_hbm.at[idx_v.at[r]], sem).wait()

    pl.run_scoped(run_scoped_sem, pltpu.SemaphoreType.DMA)


def kernel(x, on_value, off_value):
    B, S = x.shape
    offv = jnp.asarray(off_value, jnp.float32).reshape(1, 1)
    filled = pl.pallas_call(
        _fill_body,
        in_specs=[pl.BlockSpec(memory_space=pltpu.SMEM)],
        out_specs=pl.BlockSpec(memory_space=pl.ANY),
        out_shape=jax.ShapeDtypeStruct((N_WORDS,), jnp.float32),
        scratch_shapes=(
            [pltpu.VMEM((FILL_CHUNK,), jnp.float32)] * FILL_NBUF
            + [pltpu.SemaphoreType.DMA] * FILL_NBUF
        ),
    )(offv)

    xt = x.T
    on64 = jnp.full((64,), on_value, jnp.float32)
    mesh = plsc.VectorSubcoreMesh(core_axis_name="c", subcore_axis_name="s")
    f = _mpmd._mpmd_map(
        [(mesh, _scatter_body)],
        jax.ShapeDtypeStruct((N_WORDS,), jnp.float32),
        input_output_aliases={0: 0},
        compiler_params=pltpu.CompilerParams(
            use_tc_tiling_on_sc=False, needs_layout_passes=False
        ),
        scratch_types=[
            pltpu.VMEM((S, B // 32), jnp.int32),
            pltpu.VMEM((64,), jnp.float32),
            pltpu.VMEM((25, 64), jnp.int32),
        ],
    )
    out = f(filled, xt, on64)
    out5 = out.reshape(S, DEPTH // 8, B // 128, 8, 128)
    return out5.transpose(2, 4, 0, 1, 3).reshape(B, S, DEPTH)
